# baseline (speedup 1.0000x reference)
import functools

import jax
import jax.numpy as jnp
from jax.experimental import pallas as pl

_G = 2


def _moe_step(ids_ref, wts_ref, x_ref, w1gl_ref, w1gh_ref, w1ul_ref,
              w1uh_ref, w2l_ref, w2h_ref, out_ref, *, N, G):
    i = pl.program_id(0)
    x = x_ref[...]
    ids = ids_ref[...]
    wts = wts_ref[...]
    contrib = None
    for g in range(G):
        e = i * G + g
        wpe = jnp.sum(jnp.where(ids == e, wts, 0.0), axis=1)
        c = None
        for w1g_ref, w1u_ref, w2_ref in (
            (w1gl_ref, w1ul_ref, w2l_ref),
            (w1gh_ref, w1uh_ref, w2h_ref),
        ):
            gate = jax.lax.dot_general(
                x, w1g_ref[g], (((1,), (1,)), ((), ())),
                preferred_element_type=jnp.float32,
            )
            up = jax.lax.dot_general(
                x, w1u_ref[g], (((1,), (1,)), ((), ())),
                preferred_element_type=jnp.float32,
            )
            act = gate * jax.lax.logistic(gate) * up
            act = act * wpe[:, None]
            part = jax.lax.dot_general(
                act, w2_ref[g], (((1,), (1,)), ((), ())),
                preferred_element_type=jnp.float32,
            )
            c = part if c is None else c + part
        contrib = c if contrib is None else contrib + c

    @pl.when(i == 0)
    def _init():
        out_ref[...] = contrib

    @pl.when(i != 0)
    def _acc():
        out_ref[...] += contrib


def kernel(hidden_states, w1, w2, topk_weights, topk_ids):
    M, K = hidden_states.shape
    E, twoN, _ = w1.shape
    N = twoN // 2
    H = N // 2
    G = _G
    grid = (E // G,)
    out = pl.pallas_call(
        functools.partial(_moe_step, N=N, G=G),
        grid=grid,
        in_specs=[
            pl.BlockSpec(topk_ids.shape, lambda i: (0, 0)),
            pl.BlockSpec(topk_weights.shape, lambda i: (0, 0)),
            pl.BlockSpec((M, K), lambda i: (0, 0)),
            pl.BlockSpec((G, H, K), lambda i: (i, 0, 0)),
            pl.BlockSpec((G, H, K), lambda i: (i, 1, 0)),
            pl.BlockSpec((G, H, K), lambda i: (i, 2, 0)),
            pl.BlockSpec((G, H, K), lambda i: (i, 3, 0)),
            pl.BlockSpec((G, K, H), lambda i: (i, 0, 0)),
            pl.BlockSpec((G, K, H), lambda i: (i, 0, 1)),
        ],
        out_specs=pl.BlockSpec((M, K), lambda i: (0, 0)),
        out_shape=jax.ShapeDtypeStruct((M, K), hidden_states.dtype),
    )(topk_ids, topk_weights, hidden_states, w1, w1, w1, w1, w2, w2)
    return out
